# baseline (speedup 1.0000x reference)
import functools
import math

import jax
import jax.numpy as jnp
from jax.experimental import pallas as pl
from jax.experimental.pallas import tpu as pltpu

_VOCAB = 100000
_PAD = 0
_SMOOTH = 0.1
_CONF = 1.0 - _SMOOTH
_FILL = _SMOOTH / (_VOCAB - 2)
_C = _CONF * math.log(_CONF) + _SMOOTH * math.log(_FILL)

_BLOCK_ROWS = 128
_BLOCK_COLS = 4096


def _ls_kernel(y_ref, t_ref, out_ref, *, block_cols, vocab):
    k = pl.program_id(1)
    base = k * block_cols
    col = base + jax.lax.broadcasted_iota(jnp.int32, (1, block_cols), 1)
    x = jnp.where(col < vocab, y_ref[...], 0.0)

    t = t_ref[...]
    valid = (t != _PAD).astype(jnp.float32)

    s_part = jnp.sum(x, axis=1, keepdims=True)
    s_valid = jnp.sum(s_part * valid, keepdims=True)

    g = jnp.where(col == t, x, 0.0)
    g_sum = jnp.sum(jnp.sum(g, axis=1, keepdims=True) * valid, keepdims=True)

    contrib = -_FILL * s_valid - (_CONF - _FILL) * g_sum

    @pl.when(k == 0)
    def _():
        z_sum = jnp.sum(x[:, 0:1] * valid, keepdims=True)
        n_valid = jnp.sum(valid, keepdims=True)
        out_ref[0] = contrib + _FILL * z_sum + n_valid * _C

    @pl.when(k != 0)
    def _():
        out_ref[0] += contrib


def kernel(yhat, target):
    n, vocab = yhat.shape
    t2 = target.astype(jnp.int32).reshape(n, 1)
    n_row_blocks = n // _BLOCK_ROWS
    n_col_blocks = pl.cdiv(vocab, _BLOCK_COLS)
    out = pl.pallas_call(
        functools.partial(_ls_kernel, block_cols=_BLOCK_COLS, vocab=vocab),
        grid=(n_row_blocks, n_col_blocks),
        in_specs=[
            pl.BlockSpec((_BLOCK_ROWS, _BLOCK_COLS), lambda i, k: (i, k)),
            pl.BlockSpec((_BLOCK_ROWS, 1), lambda i, k: (i, 0)),
        ],
        out_specs=pl.BlockSpec((1, 1, 1), lambda i, k: (i, 0, 0)),
        out_shape=jax.ShapeDtypeStruct((n_row_blocks, 1, 1), jnp.float32),
        compiler_params=pltpu.CompilerParams(
            dimension_semantics=("parallel", "arbitrary"),
        ),
    )(yhat, t2)
    return jnp.sum(out)


# speedup vs baseline: 1.2174x; 1.2174x over previous
import functools
import math

import jax
import jax.numpy as jnp
from jax.experimental import pallas as pl
from jax.experimental.pallas import tpu as pltpu

_VOCAB = 100000
_PAD = 0
_SMOOTH = 0.1
_CONF = 1.0 - _SMOOTH
_FILL = _SMOOTH / (_VOCAB - 2)
_C = _CONF * math.log(_CONF) + _SMOOTH * math.log(_FILL)

_BLOCK_ROWS = 1024
_BLOCK_COLS = 4096


def _ls_kernel(y_ref, t_ref, out_ref, *, block_cols, vocab):
    k = pl.program_id(1)
    base = k * block_cols
    col = base + jax.lax.broadcasted_iota(jnp.int32, (1, block_cols), 1)
    x = jnp.where(col < vocab, y_ref[...], 0.0)

    t = t_ref[...]
    valid = (t != _PAD).astype(jnp.float32)

    s_part = jnp.sum(x, axis=1, keepdims=True)
    s_valid = jnp.sum(s_part * valid, keepdims=True)

    g = jnp.where(col == t, x, 0.0)
    g_sum = jnp.sum(jnp.sum(g, axis=1, keepdims=True) * valid, keepdims=True)

    contrib = -_FILL * s_valid - (_CONF - _FILL) * g_sum

    @pl.when(k == 0)
    def _():
        z_sum = jnp.sum(x[:, 0:1] * valid, keepdims=True)
        n_valid = jnp.sum(valid, keepdims=True)
        out_ref[0] = contrib + _FILL * z_sum + n_valid * _C

    @pl.when(k != 0)
    def _():
        out_ref[0] += contrib


def kernel(yhat, target):
    n, vocab = yhat.shape
    t2 = target.astype(jnp.int32).reshape(n, 1)
    n_row_blocks = n // _BLOCK_ROWS
    n_col_blocks = pl.cdiv(vocab, _BLOCK_COLS)
    out = pl.pallas_call(
        functools.partial(_ls_kernel, block_cols=_BLOCK_COLS, vocab=vocab),
        grid=(n_row_blocks, n_col_blocks),
        in_specs=[
            pl.BlockSpec((_BLOCK_ROWS, _BLOCK_COLS), lambda i, k: (i, k)),
            pl.BlockSpec((_BLOCK_ROWS, 1), lambda i, k: (i, 0)),
        ],
        out_specs=pl.BlockSpec((1, 1, 1), lambda i, k: (i, 0, 0)),
        out_shape=jax.ShapeDtypeStruct((n_row_blocks, 1, 1), jnp.float32),
        compiler_params=pltpu.CompilerParams(
            dimension_semantics=("parallel", "arbitrary"),
        ),
    )(yhat, t2)
    return jnp.sum(out)
